# baseline (speedup 1.0000x reference)
import functools

import jax
import jax.numpy as jnp
from jax import lax
from jax.experimental import pallas as pl
from jax.experimental.pallas import tpu as pltpu
from jax.experimental.pallas import tpu_sc as plsc

_B, _C, _H, _W = 4, 96, 384, 384
_K = 16
_P = _K * _K
_G = _C // 16
_CB = 24

_sc_mesh = plsc.VectorSubcoreMesh(core_axis_name="c", subcore_axis_name="s")


@functools.partial(
    pl.kernel,
    out_type=jax.ShapeDtypeStruct((_B * _P,), jnp.float32),
    mesh=_sc_mesh,
    scratch_types=[
        pltpu.VMEM((_P, _C), jnp.float32),
        pltpu.VMEM((_C,), jnp.int32),
        pltpu.VMEM((_C,), jnp.float32),
        pltpu.VMEM((_P,), jnp.float32),
        pltpu.SemaphoreType.DMA,
        pltpu.SemaphoreType.DMA,
        pltpu.SemaphoreType.DMA,
    ],
)
def _sc_patch(xt_hbm, x0_hbm, out_hbm, win_v, idx_v, val_v, zero_v, s1, s2, s3):
    wid = lax.axis_index("s") * 2 + lax.axis_index("c")

    @pl.when(wid < _B)
    def _():
        base = wid * _P
        win_dma = pltpu.async_copy(xt_hbm.at[wid], win_v, s1)
        for r in range(_P // 16):
            zero_v[pl.ds(16 * r, 16)] = jnp.zeros((16,), jnp.float32)
        zero_dma = pltpu.async_copy(zero_v, out_hbm.at[pl.ds(base, _P)], s2)
        win_dma.wait()

        init = tuple(
            [jnp.full((16,), -1.0, jnp.float32)] * _G
            + [jnp.zeros((16,), jnp.int32)] * _G
        )

        def step(j, carry):
            ms, ps = list(carry[:_G]), list(carry[_G:])
            for u in range(4):
                e = 4 * j + u
                for q in range(_G):
                    a = jnp.abs(win_v[e, pl.ds(16 * q, 16)])
                    upd = a > ms[q]
                    ms[q] = jnp.where(upd, a, ms[q])
                    ps[q] = jnp.where(upd, e, ps[q])
            return tuple(ms + ps)

        carry = lax.fori_loop(0, _P // 4, step, init)
        for q in range(_G):
            idx_v[pl.ds(16 * q, 16)] = carry[_G + q] + base

        pltpu.async_copy(x0_hbm.at[idx_v], val_v, s3).wait()
        zero_dma.wait()
        pltpu.async_copy(val_v, out_hbm.at[idx_v], s1).wait()


def _fill_body(patch_ref, out_ref):
    jc = pl.program_id(1)
    out_ref[...] = jnp.zeros_like(out_ref)

    @pl.when(jc == 0)
    def _():
        out_ref[0, 0, 0:_K, 0:_K] = patch_ref[0]


def kernel(input_):
    xw = input_[:, :, :_K, :_K].reshape(_B, _C, _P)
    xt = xw.transpose(0, 2, 1)
    x0 = xw[:, 0, :].reshape(_B * _P)
    patch = _sc_patch(xt, x0).reshape(_B, _K, _K)
    return pl.pallas_call(
        _fill_body,
        grid=(_B, _C // _CB),
        in_specs=[pl.BlockSpec((1, _K, _K), lambda b, jc: (b, 0, 0))],
        out_specs=pl.BlockSpec((1, _CB, _H, _W), lambda b, jc: (b, jc, 0, 0)),
        out_shape=jax.ShapeDtypeStruct((_B, _C, _H, _W), jnp.float32),
        compiler_params=pltpu.CompilerParams(
            dimension_semantics=("parallel", "parallel"),
        ),
    )(patch)


# speedup vs baseline: 1.1079x; 1.1079x over previous
import functools

import jax
import jax.numpy as jnp
from jax import lax
from jax.experimental import pallas as pl
from jax.experimental.pallas import tpu as pltpu
from jax.experimental.pallas import tpu_sc as plsc

_B, _C, _H, _W = 4, 96, 384, 384
_K = 16
_P = _K * _K
_G = _C // 16
_CB = 24


def _zero_body(out_ref):
    out_ref[...] = jnp.zeros_like(out_ref)


_sc_mesh = plsc.VectorSubcoreMesh(core_axis_name="c", subcore_axis_name="s")


@functools.partial(
    pl.kernel,
    out_type=jax.ShapeDtypeStruct((_B * _P,), jnp.float32),
    mesh=_sc_mesh,
    scratch_types=[
        pltpu.VMEM((_P, _C), jnp.float32),
        pltpu.VMEM((_C,), jnp.int32),
        pltpu.VMEM((_C,), jnp.float32),
        pltpu.VMEM((_P,), jnp.float32),
        pltpu.SemaphoreType.DMA,
        pltpu.SemaphoreType.DMA,
        pltpu.SemaphoreType.DMA,
    ],
)
def _sc_patch(xt_hbm, x0_hbm, out_hbm, win_v, idx_v, val_v, zero_v, s1, s2, s3):
    wid = lax.axis_index("s") * 2 + lax.axis_index("c")

    @pl.when(wid < _B)
    def _():
        base = wid * _P
        win_dma = pltpu.async_copy(xt_hbm.at[wid], win_v, s1)
        for r in range(_P // 16):
            zero_v[pl.ds(16 * r, 16)] = jnp.zeros((16,), jnp.float32)
        zero_dma = pltpu.async_copy(zero_v, out_hbm.at[pl.ds(base, _P)], s2)
        win_dma.wait()

        init = []
        for q in range(_G):
            init.append(jnp.abs(win_v[0, pl.ds(16 * q, 16)]))
        init += [jnp.zeros((16,), jnp.int32)] * _G

        def step(e, carry):
            ms, ps = carry[:_G], carry[_G:]
            out_m, out_p = [], []
            for q in range(_G):
                a = jnp.abs(win_v[e, pl.ds(16 * q, 16)])
                upd = a > ms[q]
                out_m.append(jnp.where(upd, a, ms[q]))
                out_p.append(jnp.where(upd, e, ps[q]))
            return tuple(out_m + out_p)

        carry = lax.fori_loop(1, _P, step, tuple(init))
        for q in range(_G):
            idx_v[pl.ds(16 * q, 16)] = carry[_G + q] + base

        pltpu.async_copy(x0_hbm.at[idx_v], val_v, s3).wait()
        zero_dma.wait()
        pltpu.async_copy(val_v, out_hbm.at[idx_v], s1).wait()


def _stitch_body(canvas_ref, patch_ref, out_ref):
    del canvas_ref
    out_ref[...] = jnp.zeros_like(out_ref)
    out_ref[0, 0, :, 0:_K] = patch_ref[0]


def kernel(input_):
    xw = input_[:, :, :_K, :_K].reshape(_B, _C, _P)
    xt = xw.transpose(0, 2, 1)
    x0 = xw[:, 0, :].reshape(_B * _P)
    patch = _sc_patch(xt, x0).reshape(_B, _K, _K)
    canvas = pl.pallas_call(
        _zero_body,
        grid=(_B, _C // _CB),
        out_specs=pl.BlockSpec((1, _CB, _H, _W), lambda b, jc: (b, jc, 0, 0)),
        out_shape=jax.ShapeDtypeStruct((_B, _C, _H, _W), jnp.float32),
        compiler_params=pltpu.CompilerParams(
            dimension_semantics=("parallel", "parallel"),
        ),
    )()
    return pl.pallas_call(
        _stitch_body,
        grid=(_B,),
        in_specs=[
            pl.BlockSpec(memory_space=pl.ANY),
            pl.BlockSpec((1, _K, _K), lambda b: (b, 0, 0)),
        ],
        out_specs=pl.BlockSpec((1, 1, _K, _W), lambda b: (b, 0, 0, 0)),
        out_shape=jax.ShapeDtypeStruct((_B, _C, _H, _W), jnp.float32),
        input_output_aliases={0: 0},
    )(canvas, patch)


# speedup vs baseline: 1.1267x; 1.0170x over previous
import functools

import jax
import jax.numpy as jnp
from jax import lax
from jax.experimental import pallas as pl
from jax.experimental.pallas import tpu as pltpu
from jax.experimental.pallas import tpu_sc as plsc

_B, _C, _H, _W = 4, 96, 384, 384
_K = 16
_P = _K * _K
_G = _C // 16
_CB = 24


def _zero_body(out_ref):
    out_ref[...] = jnp.zeros_like(out_ref)


_sc_mesh = plsc.VectorSubcoreMesh(
    core_axis_name="c", subcore_axis_name="s", num_cores=1
)


@functools.partial(
    pl.kernel,
    out_type=jax.ShapeDtypeStruct((_B * _P,), jnp.float32),
    mesh=_sc_mesh,
    scratch_types=[
        pltpu.VMEM((_P, _C), jnp.float32),
        pltpu.VMEM((_C,), jnp.int32),
        pltpu.VMEM((_C,), jnp.float32),
        pltpu.VMEM((_P,), jnp.float32),
        pltpu.SemaphoreType.DMA,
        pltpu.SemaphoreType.DMA,
        pltpu.SemaphoreType.DMA,
    ],
)
def _sc_patch(xt_hbm, x0_hbm, out_hbm, win_v, idx_v, val_v, zero_v, s1, s2, s3):
    wid = lax.axis_index("s")

    @pl.when(wid < _B)
    def _():
        base = wid * _P
        win_dma = pltpu.async_copy(xt_hbm.at[wid], win_v, s1)
        for r in range(_P // 16):
            zero_v[pl.ds(16 * r, 16)] = jnp.zeros((16,), jnp.float32)
        zero_dma = pltpu.async_copy(zero_v, out_hbm.at[pl.ds(base, _P)], s2)
        win_dma.wait()

        init = []
        for q in range(_G):
            init.append(jnp.abs(win_v[0, pl.ds(16 * q, 16)]))
        init += [jnp.zeros((16,), jnp.int32)] * _G

        def step(e, carry):
            ms, ps = carry[:_G], carry[_G:]
            out_m, out_p = [], []
            for q in range(_G):
                a = jnp.abs(win_v[e, pl.ds(16 * q, 16)])
                upd = a > ms[q]
                out_m.append(jnp.where(upd, a, ms[q]))
                out_p.append(jnp.where(upd, e, ps[q]))
            return tuple(out_m + out_p)

        carry = lax.fori_loop(1, _P, step, tuple(init))
        for q in range(_G):
            idx_v[pl.ds(16 * q, 16)] = carry[_G + q] + base

        pltpu.async_copy(x0_hbm.at[idx_v], val_v, s3).wait()
        zero_dma.wait()
        pltpu.async_copy(val_v, out_hbm.at[idx_v], s1).wait()


def _stitch_body(canvas_ref, patch_ref, out_ref):
    del canvas_ref
    out_ref[...] = jnp.zeros_like(out_ref)
    out_ref[0, 0, :, 0:_K] = patch_ref[0]


def kernel(input_):
    xw = input_[:, :, :_K, :_K].reshape(_B, _C, _P)
    xt = xw.transpose(0, 2, 1)
    x0 = xw[:, 0, :].reshape(_B * _P)
    patch = _sc_patch(xt, x0).reshape(_B, _K, _K)
    canvas = pl.pallas_call(
        _zero_body,
        grid=(_B, _C // _CB),
        out_specs=pl.BlockSpec((1, _CB, _H, _W), lambda b, jc: (b, jc, 0, 0)),
        out_shape=jax.ShapeDtypeStruct((_B, _C, _H, _W), jnp.float32),
        compiler_params=pltpu.CompilerParams(
            dimension_semantics=("parallel", "parallel"),
        ),
    )()
    return pl.pallas_call(
        _stitch_body,
        grid=(_B,),
        in_specs=[
            pl.BlockSpec(memory_space=pl.ANY),
            pl.BlockSpec((1, _K, _K), lambda b: (b, 0, 0)),
        ],
        out_specs=pl.BlockSpec((1, 1, _K, _W), lambda b: (b, 0, 0, 0)),
        out_shape=jax.ShapeDtypeStruct((_B, _C, _H, _W), jnp.float32),
        input_output_aliases={0: 0},
    )(canvas, patch)
